# Initial kernel scaffold; baseline (speedup 1.0000x reference)
import functools

import jax
import jax.numpy as jnp
from jax import lax
from jax.experimental import pallas as pl
from jax.experimental.pallas import tpu as pltpu
from jax.experimental.pallas import tpu_sc as plsc

PW = 16


def _sc_gather(xflat, ppad, nidx, rows_per_worker, chunk):
    rows, c = nidx.shape[0], xflat.shape[1]
    nw = 32
    assert rows == nw * rows_per_worker
    assert rows_per_worker % chunk == 0 and chunk % 8 == 0 and chunk <= 128
    nchunk = rows_per_worker // chunk

    mesh = plsc.VectorSubcoreMesh(core_axis_name="c", subcore_axis_name="s")

    @functools.partial(
        pl.kernel,
        out_type=[
            jax.ShapeDtypeStruct((rows, c), jnp.float32),
            jax.ShapeDtypeStruct((rows, PW), jnp.float32),
        ],
        mesh=mesh,
        scratch_types=[
            pltpu.VMEM((rows_per_worker,), jnp.int32),
            pltpu.VMEM((chunk, c), jnp.float32),
            pltpu.VMEM((chunk, PW), jnp.float32),
            pltpu.SemaphoreType.DMA,
            pltpu.SemaphoreType.DMA,
        ],
    )
    def k(xflat_hbm, ppad_hbm, nidx_hbm, nbrx_hbm, nbrp_hbm,
          idx_v, xrows_v, prows_v, sem1, sem2):
        wid = lax.axis_index("s") * 2 + lax.axis_index("c")
        base = wid * rows_per_worker
        pltpu.sync_copy(nidx_hbm.at[pl.ds(base, rows_per_worker)], idx_v)

        def body(g, carry):
            off = pl.multiple_of(g * chunk, chunk)
            idx_c = idx_v.at[pl.ds(off, chunk)]
            cp1 = pltpu.async_copy(xflat_hbm.at[idx_c], xrows_v, sem1)
            cp2 = pltpu.async_copy(ppad_hbm.at[idx_c], prows_v, sem2)
            cp1.wait()
            cp2.wait()
            pltpu.sync_copy(xrows_v, nbrx_hbm.at[pl.ds(base + off, chunk)])
            pltpu.sync_copy(prows_v, nbrp_hbm.at[pl.ds(base + off, chunk)])
            return carry

        lax.fori_loop(0, nchunk, body, 0)

    return k(xflat, ppad, nidx)


def _tc_body(nbrp_ref, nbrx_ref, pp_ref, a_ref, w1p_ref, b1_ref, g1_ref,
             be1_ref, w2t_ref, b2_ref, g2_ref, be2_ref, out_ref):
    b, tn, kc = nbrx_ref.shape
    hid = w1p_ref.shape[1]
    out = w2t_ref.shape[1]
    c = out_ref.shape[2]
    k = kc // c

    nbrp = nbrp_ref[...].reshape(b * tn, k * PW)
    pp = pp_ref[...].reshape(b * tn, PW)
    h = (jnp.dot(pp, a_ref[...], preferred_element_type=jnp.float32)
         - jnp.dot(nbrp, w1p_ref[...], preferred_element_type=jnp.float32)
         + b1_ref[...])
    h3 = h.reshape(b, tn, hid)
    mu = jnp.mean(h3, axis=(0, 2), keepdims=True)
    ctr = h3 - mu
    var = jnp.mean(ctr * ctr, axis=(0, 2), keepdims=True)
    hn = ctr * lax.rsqrt(var + 1e-5)
    hn = hn * g1_ref[...][None] + be1_ref[...][None]
    hr = jnp.maximum(hn, 0.0).reshape(b * tn, hid)

    o = jnp.dot(hr, w2t_ref[...], preferred_element_type=jnp.float32) + b2_ref[...]
    o3 = o.reshape(b, tn, out)
    mu2 = jnp.mean(o3, axis=(0, 2), keepdims=True)
    ctr2 = o3 - mu2
    var2 = jnp.mean(ctr2 * ctr2, axis=(0, 2), keepdims=True)
    on = ctr2 * lax.rsqrt(var2 + 1e-5)
    on = on * g2_ref[...][None] + be2_ref[...][None]
    y1 = jnp.maximum(on, 0.0)

    y2 = nbrx_ref[...]
    acc = y1[:, :, 0:c] * y2[:, :, 0:c]
    for j in range(1, k):
        acc = acc + y1[:, :, j * c:(j + 1) * c] * y2[:, :, j * c:(j + 1) * c]
    out_ref[...] = acc


def _tc_mlp(nbrp3, nbrx3, ppad3, a, w1p, b1r, g1c, be1c, w2t, b2r, g2c, be2c,
            tn, interpret=False):
    b, n, kc = nbrx3.shape
    kpw = nbrp3.shape[2]
    hid = w1p.shape[1]
    out = w2t.shape[1]
    c = out // (kpw // PW)
    grid = (n // tn,)
    return pl.pallas_call(
        _tc_body,
        grid=grid,
        in_specs=[
            pl.BlockSpec((b, tn, kpw), lambda i: (0, i, 0)),
            pl.BlockSpec((b, tn, kc), lambda i: (0, i, 0)),
            pl.BlockSpec((b, tn, PW), lambda i: (0, i, 0)),
            pl.BlockSpec((PW, hid), lambda i: (0, 0)),
            pl.BlockSpec((kpw, hid), lambda i: (0, 0)),
            pl.BlockSpec((1, hid), lambda i: (0, 0)),
            pl.BlockSpec((tn, 1), lambda i: (i, 0)),
            pl.BlockSpec((tn, 1), lambda i: (i, 0)),
            pl.BlockSpec((hid, out), lambda i: (0, 0)),
            pl.BlockSpec((1, out), lambda i: (0, 0)),
            pl.BlockSpec((tn, 1), lambda i: (i, 0)),
            pl.BlockSpec((tn, 1), lambda i: (i, 0)),
        ],
        out_specs=pl.BlockSpec((b, tn, c), lambda i: (0, i, 0)),
        out_shape=jax.ShapeDtypeStruct((b, n, c), jnp.float32),
        interpret=interpret,
    )(nbrp3, nbrx3, ppad3, a, w1p, b1r, g1c, be1c, w2t, b2r, g2c, be2c)


def kernel(x, points, indices, W1, b1, g1, be1, W2, b2, g2, be2):
    b, n, c = x.shape
    k = indices.shape[2]
    hid = W1.shape[0]
    out = W2.shape[0]

    xflat = x.reshape(b * n, c)
    ppad = jnp.pad(points, ((0, 0), (0, 0), (0, PW - points.shape[2]))) \
        .reshape(b * n, PW)
    nidx = (indices.astype(jnp.int32)
            + (jnp.arange(b, dtype=jnp.int32) * n)[:, None, None]).reshape(-1)

    w1r = W1.reshape(hid, k, points.shape[2])
    w1pad = jnp.pad(w1r, ((0, 0), (0, 0), (0, PW - points.shape[2])))
    w1p = w1pad.transpose(1, 2, 0).reshape(k * PW, hid)
    a = w1p.reshape(k, PW, hid).sum(axis=0)
    w2t = W2.T
    b1r = b1.reshape(1, hid)
    b2r = b2.reshape(1, out)
    g1c = g1.reshape(n, 1)
    be1c = be1.reshape(n, 1)
    g2c = g2.reshape(n, 1)
    be2c = be2.reshape(n, 1)

    rows = b * n * k
    nbrx, nbrp = _sc_gather(xflat, ppad, nidx,
                            rows_per_worker=rows // 32, chunk=80)
    nbrx3 = nbrx.reshape(b, n, k * c)
    nbrp3 = nbrp.reshape(b, n, k * PW)

    tn = 400 if n % 400 == 0 else n
    res = _tc_mlp(nbrp3, nbrx3, ppad.reshape(b, n, PW), a, w1p, b1r, g1c,
                  be1c, w2t, b2r, g2c, be2c, tn)
    return (res, points, indices)


# speedup vs baseline: 16.2365x; 16.2365x over previous
import functools

import jax
import jax.numpy as jnp
from jax import lax
from jax.experimental import pallas as pl
from jax.experimental.pallas import tpu as pltpu
from jax.experimental.pallas import tpu_sc as plsc

PW = 16


def _sc_gather(xflat, ppad, nidx, rows_per_worker, chunk):
    rows, c = nidx.shape[0], xflat.shape[1]
    nw = 32
    assert rows == nw * rows_per_worker
    assert rows_per_worker % chunk == 0 and chunk % 8 == 0 and chunk <= 128
    nchunk = rows_per_worker // chunk

    mesh = plsc.VectorSubcoreMesh(core_axis_name="c", subcore_axis_name="s")

    @functools.partial(
        pl.kernel,
        out_type=[
            jax.ShapeDtypeStruct((rows, c), jnp.float32),
            jax.ShapeDtypeStruct((rows, PW), jnp.float32),
        ],
        mesh=mesh,
        compiler_params=pltpu.CompilerParams(use_tc_tiling_on_sc=False),
        scratch_types=[
            pltpu.VMEM((rows_per_worker,), jnp.int32),
            pltpu.VMEM((chunk, c), jnp.float32),
            pltpu.VMEM((chunk, PW), jnp.float32),
            pltpu.SemaphoreType.DMA,
            pltpu.SemaphoreType.DMA,
        ],
    )
    def k(xflat_hbm, ppad_hbm, nidx_hbm, nbrx_hbm, nbrp_hbm,
          idx_v, xrows_v, prows_v, sem1, sem2):
        wid = lax.axis_index("s") * 2 + lax.axis_index("c")
        base = wid * rows_per_worker
        pltpu.sync_copy(nidx_hbm.at[pl.ds(base, rows_per_worker)], idx_v)

        def body(g, carry):
            off = pl.multiple_of(g * chunk, chunk)
            idx_c = idx_v.at[pl.ds(off, chunk)]
            cp1 = pltpu.async_copy(xflat_hbm.at[idx_c], xrows_v, sem1)
            cp2 = pltpu.async_copy(ppad_hbm.at[idx_c], prows_v, sem2)
            cp1.wait()
            cp2.wait()
            pltpu.sync_copy(xrows_v, nbrx_hbm.at[pl.ds(base + off, chunk)])
            pltpu.sync_copy(prows_v, nbrp_hbm.at[pl.ds(base + off, chunk)])
            return carry

        lax.fori_loop(0, nchunk, body, 0)

    return k(xflat, ppad, nidx)


def _tc_body(nbrp_ref, nbrx_ref, pp_ref, a_ref, w1p_ref, b1_ref, g1_ref,
             be1_ref, w2t_ref, b2_ref, g2_ref, be2_ref, out_ref):
    b, tn, kc = nbrx_ref.shape
    hid = w1p_ref.shape[1]
    out = w2t_ref.shape[1]
    c = out_ref.shape[2]
    k = kc // c

    nbrp = nbrp_ref[...].reshape(b * tn, k * PW)
    pp = pp_ref[...].reshape(b * tn, PW)
    h = (jnp.dot(pp, a_ref[...], preferred_element_type=jnp.float32)
         - jnp.dot(nbrp, w1p_ref[...], preferred_element_type=jnp.float32)
         + b1_ref[...])
    h3 = h.reshape(b, tn, hid)
    mu = jnp.mean(h3, axis=(0, 2), keepdims=True)
    ctr = h3 - mu
    var = jnp.mean(ctr * ctr, axis=(0, 2), keepdims=True)
    hn = ctr * lax.rsqrt(var + 1e-5)
    hn = hn * g1_ref[...][None] + be1_ref[...][None]
    hr = jnp.maximum(hn, 0.0).reshape(b * tn, hid)

    o = jnp.dot(hr, w2t_ref[...], preferred_element_type=jnp.float32) + b2_ref[...]
    o3 = o.reshape(b, tn, out)
    mu2 = jnp.mean(o3, axis=(0, 2), keepdims=True)
    ctr2 = o3 - mu2
    var2 = jnp.mean(ctr2 * ctr2, axis=(0, 2), keepdims=True)
    on = ctr2 * lax.rsqrt(var2 + 1e-5)
    on = on * g2_ref[...][None] + be2_ref[...][None]
    y1 = jnp.maximum(on, 0.0)

    y2 = nbrx_ref[...]
    acc = y1[:, :, 0:c] * y2[:, :, 0:c]
    for j in range(1, k):
        acc = acc + y1[:, :, j * c:(j + 1) * c] * y2[:, :, j * c:(j + 1) * c]
    out_ref[...] = acc


def _tc_mlp(nbrp3, nbrx3, ppad3, a, w1p, b1r, g1c, be1c, w2t, b2r, g2c, be2c,
            tn):
    b, n, kc = nbrx3.shape
    kpw = nbrp3.shape[2]
    hid = w1p.shape[1]
    out = w2t.shape[1]
    c = out // (kpw // PW)
    grid = (n // tn,)
    return pl.pallas_call(
        _tc_body,
        grid=grid,
        in_specs=[
            pl.BlockSpec((b, tn, kpw), lambda i: (0, i, 0)),
            pl.BlockSpec((b, tn, kc), lambda i: (0, i, 0)),
            pl.BlockSpec((b, tn, PW), lambda i: (0, i, 0)),
            pl.BlockSpec((PW, hid), lambda i: (0, 0)),
            pl.BlockSpec((kpw, hid), lambda i: (0, 0)),
            pl.BlockSpec((1, hid), lambda i: (0, 0)),
            pl.BlockSpec((tn, 1), lambda i: (i, 0)),
            pl.BlockSpec((tn, 1), lambda i: (i, 0)),
            pl.BlockSpec((hid, out), lambda i: (0, 0)),
            pl.BlockSpec((1, out), lambda i: (0, 0)),
            pl.BlockSpec((tn, 1), lambda i: (i, 0)),
            pl.BlockSpec((tn, 1), lambda i: (i, 0)),
        ],
        out_specs=pl.BlockSpec((b, tn, c), lambda i: (0, i, 0)),
        out_shape=jax.ShapeDtypeStruct((b, n, c), jnp.float32),
    )(nbrp3, nbrx3, ppad3, a, w1p, b1r, g1c, be1c, w2t, b2r, g2c, be2c)


def kernel(x, points, indices, W1, b1, g1, be1, W2, b2, g2, be2):
    b, n, c = x.shape
    k = indices.shape[2]
    hid = W1.shape[0]
    out = W2.shape[0]

    xflat = x.reshape(b * n, c)
    ppad = jnp.pad(points, ((0, 0), (0, 0), (0, PW - points.shape[2]))) \
        .reshape(b * n, PW)
    nidx = (indices.astype(jnp.int32)
            + (jnp.arange(b, dtype=jnp.int32) * n)[:, None, None]).reshape(-1)

    w1r = W1.reshape(hid, k, points.shape[2])
    w1pad = jnp.pad(w1r, ((0, 0), (0, 0), (0, PW - points.shape[2])))
    w1p = w1pad.transpose(1, 2, 0).reshape(k * PW, hid)
    a = w1p.reshape(k, PW, hid).sum(axis=0)
    w2t = W2.T
    b1r = b1.reshape(1, hid)
    b2r = b2.reshape(1, out)
    g1c = g1.reshape(n, 1)
    be1c = be1.reshape(n, 1)
    g2c = g2.reshape(n, 1)
    be2c = be2.reshape(n, 1)

    rows = b * n * k
    nbrx, nbrp = _sc_gather(xflat, ppad, nidx,
                            rows_per_worker=rows // 32, chunk=80)
    nbrx3 = nbrx.reshape(b, n, k * c)
    nbrp3 = nbrp.reshape(b, n, k * PW)

    tn = 400 if n % 400 == 0 else n
    res = _tc_mlp(nbrp3, nbrx3, ppad.reshape(b, n, PW), a, w1p, b1r, g1c,
                  be1c, w2t, b2r, g2c, be2c, tn)
    return (res, points, indices)


# speedup vs baseline: 26.1145x; 1.6084x over previous
import functools

import jax
import jax.numpy as jnp
from jax import lax
from jax.experimental import pallas as pl
from jax.experimental.pallas import tpu as pltpu
from jax.experimental.pallas import tpu_sc as plsc

PW = 16


def _sc_gather(xflat, ppad, xidx, pidx, rows_per_worker):
    rows, c = xidx.shape[0], xflat.shape[1]
    nw = 32
    assert rows == nw * rows_per_worker
    chunk = 128
    nfull = rows_per_worker // chunk
    tail = rows_per_worker - nfull * chunk
    assert nfull % 2 == 0 and tail % 8 == 0 and tail < chunk

    mesh = plsc.VectorSubcoreMesh(core_axis_name="c", subcore_axis_name="s")

    @functools.partial(
        pl.kernel,
        out_type=[
            jax.ShapeDtypeStruct((rows, c), jnp.float32),
            jax.ShapeDtypeStruct((rows, PW), jnp.float32),
        ],
        mesh=mesh,
        compiler_params=pltpu.CompilerParams(use_tc_tiling_on_sc=False),
        scratch_types=[
            pltpu.VMEM((rows_per_worker,), jnp.int32),
            pltpu.VMEM((rows_per_worker,), jnp.int32),
            pltpu.VMEM((chunk, c), jnp.float32),
            pltpu.VMEM((chunk, c), jnp.float32),
            pltpu.VMEM((chunk, PW), jnp.float32),
            pltpu.VMEM((chunk, PW), jnp.float32),
            pltpu.SemaphoreType.DMA,
            pltpu.SemaphoreType.DMA,
            pltpu.SemaphoreType.DMA,
            pltpu.SemaphoreType.DMA,
        ],
    )
    def k(xflat_hbm, ppad_hbm, xidx_hbm, pidx_hbm, nbrx_hbm, nbrp_hbm,
          xidx_v, pidx_v, xr0, xr1, pr0, pr1, sx0, sx1, sp0, sp1):
        wid = lax.axis_index("s") * 2 + lax.axis_index("c")
        base = wid * rows_per_worker
        pltpu.sync_copy(xidx_hbm.at[pl.ds(base, rows_per_worker)], xidx_v)
        pltpu.sync_copy(pidx_hbm.at[pl.ds(base, rows_per_worker)], pidx_v)

        def start(g, xr, pr, sx, sp, nrows=chunk):
            off = pl.multiple_of(g * chunk, chunk)
            pltpu.async_copy(
                xflat_hbm.at[xidx_v.at[pl.ds(off, nrows)]],
                xr.at[pl.ds(0, nrows)], sx)
            pltpu.async_copy(
                ppad_hbm.at[pidx_v.at[pl.ds(off, nrows)]],
                pr.at[pl.ds(0, nrows)], sp)

        def drain(xr, pr, sx, sp, nrows=chunk):
            pltpu.make_async_copy(
                xflat_hbm.at[pl.ds(0, nrows)], xr.at[pl.ds(0, nrows)],
                sx).wait()
            pltpu.make_async_copy(
                ppad_hbm.at[pl.ds(0, nrows)], pr.at[pl.ds(0, nrows)],
                sp).wait()

        def write(g, xr, pr, nrows=chunk):
            off = pl.multiple_of(g * chunk, chunk)
            pltpu.sync_copy(xr.at[pl.ds(0, nrows)],
                            nbrx_hbm.at[pl.ds(base + off, nrows)])
            pltpu.sync_copy(pr.at[pl.ds(0, nrows)],
                            nbrp_hbm.at[pl.ds(base + off, nrows)])

        start(0, xr0, pr0, sx0, sp0)

        def body(go, carry):
            g0 = pl.multiple_of(go * 2, 2)
            start(g0 + 1, xr1, pr1, sx1, sp1)
            drain(xr0, pr0, sx0, sp0)
            write(g0, xr0, pr0)
            start(g0 + 2, xr0, pr0, sx0, sp0)
            drain(xr1, pr1, sx1, sp1)
            write(g0 + 1, xr1, pr1)
            return carry

        lax.fori_loop(0, nfull // 2 - 1, body, 0)
        g0 = nfull - 2
        start(g0 + 1, xr1, pr1, sx1, sp1)
        drain(xr0, pr0, sx0, sp0)
        write(g0, xr0, pr0)
        if tail:
            start(nfull, xr0, pr0, sx0, sp0, nrows=tail)
        drain(xr1, pr1, sx1, sp1)
        write(g0 + 1, xr1, pr1)
        if tail:
            drain(xr0, pr0, sx0, sp0, nrows=tail)
            write(nfull, xr0, pr0, nrows=tail)

    return k(xflat, ppad, xidx, pidx)


def _tc_body(nbrp_ref, nbrx_ref, pp_ref, a_ref, w1p_ref, b1_ref, g1_ref,
             be1_ref, w2_ref, b2_ref, g2_ref, be2_ref, out_ref):
    b, kk, tn, c = nbrx_ref.shape
    hid = w1p_ref.shape[1]
    out = w2_ref.shape[0]

    nbrp = nbrp_ref[...].reshape(b * tn, kk * PW)
    pp = pp_ref[...].reshape(b * tn, PW)
    h = (jnp.dot(pp, a_ref[...], preferred_element_type=jnp.float32)
         - jnp.dot(nbrp, w1p_ref[...], preferred_element_type=jnp.float32)
         + b1_ref[...])
    h3 = h.reshape(b, tn, hid)
    mu = jnp.mean(h3, axis=(0, 2), keepdims=True)
    ctr = h3 - mu
    var = jnp.mean(ctr * ctr, axis=(0, 2), keepdims=True)
    hn = ctr * lax.rsqrt(var + 1e-5)
    hn = hn * g1_ref[...][None] + be1_ref[...][None]
    hr = jnp.maximum(hn, 0.0).reshape(b * tn, hid)

    o = lax.dot_general(hr, w2_ref[...], (((1,), (1,)), ((), ())),
                        preferred_element_type=jnp.float32) + b2_ref[...]
    o3 = o.reshape(b, tn, out)
    mu2 = jnp.mean(o3, axis=(0, 2), keepdims=True)
    var2 = jnp.mean((o3 - mu2) * (o3 - mu2), axis=(0, 2), keepdims=True)
    rs2 = lax.rsqrt(var2 + 1e-5)
    g2b = g2_ref[...][None]
    be2b = be2_ref[...][None]

    acc = jnp.zeros((b, tn, c), jnp.float32)
    for j in range(kk):
        oj = (o3[:, :, j * c:(j + 1) * c] - mu2) * rs2 * g2b + be2b
        acc = acc + jnp.maximum(oj, 0.0) * nbrx_ref[:, j]
    out_ref[...] = acc


def _tc_mlp(nbrp3, nbrx4, ppad3, a, w1p, b1r, g1c, be1c, w2, b2r, g2c, be2c,
            tn):
    b, kk, n, c = nbrx4.shape
    kpw = nbrp3.shape[2]
    hid = w1p.shape[1]
    out = w2.shape[0]
    grid = (n // tn,)
    return pl.pallas_call(
        _tc_body,
        grid=grid,
        in_specs=[
            pl.BlockSpec((b, tn, kpw), lambda i: (0, i, 0)),
            pl.BlockSpec((b, kk, tn, c), lambda i: (0, 0, i, 0)),
            pl.BlockSpec((b, tn, PW), lambda i: (0, i, 0)),
            pl.BlockSpec((PW, hid), lambda i: (0, 0)),
            pl.BlockSpec((kpw, hid), lambda i: (0, 0)),
            pl.BlockSpec((1, hid), lambda i: (0, 0)),
            pl.BlockSpec((tn, 1), lambda i: (i, 0)),
            pl.BlockSpec((tn, 1), lambda i: (i, 0)),
            pl.BlockSpec((out, hid), lambda i: (0, 0)),
            pl.BlockSpec((1, out), lambda i: (0, 0)),
            pl.BlockSpec((tn, 1), lambda i: (i, 0)),
            pl.BlockSpec((tn, 1), lambda i: (i, 0)),
        ],
        out_specs=pl.BlockSpec((b, tn, c), lambda i: (0, i, 0)),
        out_shape=jax.ShapeDtypeStruct((b, n, c), jnp.float32),
    )(nbrp3, nbrx4, ppad3, a, w1p, b1r, g1c, be1c, w2, b2r, g2c, be2c)


def kernel(x, points, indices, W1, b1, g1, be1, W2, b2, g2, be2):
    b, n, c = x.shape
    k = indices.shape[2]
    hid = W1.shape[0]
    out = W2.shape[0]

    xflat = x.reshape(b * n, c)
    ppad = jnp.pad(points, ((0, 0), (0, 0), (0, PW - points.shape[2]))) \
        .reshape(b * n, PW)
    boff = (jnp.arange(b, dtype=jnp.int32) * n)
    idx32 = indices.astype(jnp.int32)
    xidx = (idx32.transpose(0, 2, 1) + boff[:, None, None]).reshape(-1)
    pidx = (idx32 + boff[:, None, None]).reshape(-1)

    w1r = W1.reshape(hid, k, points.shape[2])
    w1pad = jnp.pad(w1r, ((0, 0), (0, 0), (0, PW - points.shape[2])))
    w1p = w1pad.transpose(1, 2, 0).reshape(k * PW, hid)
    a = w1p.reshape(k, PW, hid).sum(axis=0)
    b1r = b1.reshape(1, hid)
    b2r = b2.reshape(1, out)
    g1c = g1.reshape(n, 1)
    be1c = be1.reshape(n, 1)
    g2c = g2.reshape(n, 1)
    be2c = be2.reshape(n, 1)

    rows = b * n * k
    nbrx, nbrp = _sc_gather(xflat, ppad, xidx, pidx,
                            rows_per_worker=rows // 32)
    nbrx4 = nbrx.reshape(b, k, n, c)
    nbrp3 = nbrp.reshape(b, n, k * PW)

    tn = 400 if n % 400 == 0 else n
    res = _tc_mlp(nbrp3, nbrx4, ppad.reshape(b, n, PW), a, w1p, b1r, g1c,
                  be1c, W2, b2r, g2c, be2c, tn)
    return (res, points, indices)
